# baseline (speedup 1.0000x reference)
import functools

import jax
import jax.numpy as jnp
from jax import lax
from jax.experimental import pallas as pl
from jax.experimental.pallas import tpu as pltpu
from jax.experimental.pallas import tpu_sc as plsc

_N = 25088
_D = 512
_C = 32
_B = 3136
_K = 8
_INV_TEMP = 1.0 / 0.07
_NEG = -1e30

_NW = 32
_RPW = _N // _NW
_CH = 112
_NCH = _RPW // _CH


def _sc_segsum_body(s1_hbm, lab_hbm, zero_hbm, out_hbm, lab_v, rows_v, acc_v):
    cid = lax.axis_index("c")
    sid = lax.axis_index("s")
    wid = cid * 16 + sid

    pltpu.sync_copy(zero_hbm, acc_v)
    pltpu.sync_copy(lab_hbm.at[wid], lab_v)
    for ch in range(_NCH):
        pltpu.sync_copy(
            s1_hbm.at[pl.ds(wid * _RPW + ch * _CH, _CH), :], rows_v)

        def _grp(g, carry):
            lab16 = lab_v[ch, pl.ds(g * 16, 16)]
            for k in range(16):
                lab_r = lab16[k]
                for dc in range(_D // 16):
                    sl = pl.ds(dc * 16, 16)
                    plsc.addupdate(acc_v.at[lab_r, sl],
                                   rows_v[g * 16 + k, sl])
            return carry

        lax.fori_loop(0, _CH // 16, _grp, 0, unroll=False)

    pltpu.sync_copy(acc_v, out_hbm.at[wid])


_sc_segsum = pl.kernel(
    _sc_segsum_body,
    out_type=jax.ShapeDtypeStruct((_NW, _C, _D), jnp.float32),
    mesh=plsc.VectorSubcoreMesh(core_axis_name="c", subcore_axis_name="s"),
    scratch_types=[
        pltpu.VMEM((_NCH, _CH), jnp.int32),
        pltpu.VMEM((_CH, _D), jnp.float32),
        pltpu.VMEM((_C, _D), jnp.float32),
    ],
)


def _fused(s1_ref, laba_ref, s2_ref, labb_ref, sim_ref, part_ref, out_ref,
           cache_ref, cnt_ref, cent_ref, bias_ref):
    i = pl.program_id(0)

    @pl.when(i < _K)
    def _phase_a():
        x = s1_ref[...]
        lab = laba_ref[0, 0, :]
        oh_t = (lax.broadcasted_iota(jnp.int32, (_C, _B), 0)
                == lab[None, :]).astype(jnp.float32)
        pcnt = jnp.sum(oh_t, axis=1, keepdims=True)

        cache_ref[pl.ds(i * _B, _B), :] = x.astype(jnp.bfloat16)

        @pl.when(i == 0)
        def _init():
            cnt_ref[...] = pcnt

        @pl.when(i != 0)
        def _acc():
            cnt_ref[...] += pcnt

    @pl.when(i >= _K)
    def _phase_b():
        j = i - _K

        @pl.when(i == _K)
        def _finalize():
            cnt = cnt_ref[...]
            recip = 1.0 / jnp.maximum(cnt, 1.0)
            sums = jnp.sum(part_ref[...], axis=0)
            cent_ref[...] = (sums * recip).astype(jnp.bfloat16)
            bias_ref[...] = jnp.where(cnt > 0.0, 0.0, _NEG)

        cent = cent_ref[...]
        bias = bias_ref[...]
        lab = labb_ref[0, 0, :]
        oh_t = (lax.broadcasted_iota(jnp.int32, (_C, _B), 0)
                == lab[None, :])

        def loss_of(x):
            lg = lax.dot_general(cent, x, (((1,), (1,)), ((), ())),
                                 preferred_element_type=jnp.float32)
            lg = lg * _INV_TEMP + bias
            m = jnp.max(lg, axis=0, keepdims=True)
            lse = jnp.log(jnp.sum(jnp.exp(lg - m), axis=0)) + m[0]
            picked = jnp.sum(jnp.where(oh_t, lg, 0.0), axis=0)
            return lse - picked

        x1 = cache_ref[pl.ds(j * _B, _B), :]
        x2 = s2_ref[...].astype(jnp.bfloat16)
        loss = loss_of(x1) + loss_of(x2)
        ones_row = jnp.full((1, 64), 1.0 / 64.0, dtype=jnp.float32)
        w = lax.dot_general(ones_row, sim_ref[0],
                            (((1,), (1,)), ((), ())),
                            preferred_element_type=jnp.float32)[0]
        part = jnp.sum(w * loss) * (0.25 / _N)

        @pl.when(i == _K)
        def _out_init():
            out_ref[0, 0] = part

        @pl.when(i != _K)
        def _out_acc():
            out_ref[0, 0] += part


def kernel(S1, S2, segmentation_map, similarity_matrix):
    labels_sc = segmentation_map.reshape(_NW, _NCH, _CH)
    partials = _sc_segsum(S1, labels_sc, jnp.zeros((_C, _D), jnp.float32))

    labels_a = segmentation_map.reshape(_K, 1, _B)

    out = pl.pallas_call(
        _fused,
        grid=(2 * _K,),
        in_specs=[
            pl.BlockSpec((_B, _D), lambda i: (jnp.minimum(i, _K - 1), 0)),
            pl.BlockSpec((1, 1, _B),
                         lambda i: (jnp.minimum(i, _K - 1), 0, 0)),
            pl.BlockSpec((_B, _D), lambda i: (jnp.maximum(i - _K, 0), 0)),
            pl.BlockSpec((1, 1, _B),
                         lambda i: (jnp.maximum(i - _K, 0), 0, 0)),
            pl.BlockSpec((1, _B, 64),
                         lambda i: (jnp.maximum(i - _K, 0), 0, 0)),
            pl.BlockSpec(memory_space=pltpu.VMEM),
        ],
        out_specs=pl.BlockSpec(memory_space=pltpu.SMEM),
        out_shape=jax.ShapeDtypeStruct((1, 1), jnp.float32),
        scratch_shapes=[
            pltpu.VMEM((_N, _D), jnp.bfloat16),
            pltpu.VMEM((_C, 1), jnp.float32),
            pltpu.VMEM((_C, _D), jnp.bfloat16),
            pltpu.VMEM((_C, 1), jnp.float32),
        ],
        compiler_params=pltpu.CompilerParams(
            dimension_semantics=("arbitrary",)),
    )(S1, labels_a, S2, labels_a, similarity_matrix, partials)

    return out[0, 0]


# speedup vs baseline: 3.3807x; 3.3807x over previous
import jax
import jax.numpy as jnp
from jax import lax
from jax.experimental import pallas as pl
from jax.experimental.pallas import tpu as pltpu

_N = 25088
_D = 512
_C = 32
_BA = 3136
_KA = 8
_BB = 3136
_KB = 8
_INV_TEMP = 1.0 / 0.07
_NEG = -1e30


def _fused(s1_ref, lab_ref, s2_ref, sim_ref, out_ref,
           cache_ref, sums_ref, cnt_ref, cent_ref, bias_ref):
    i = pl.program_id(0)

    @pl.when(i < _KA)
    def _phase_a():
        x = s1_ref[...]
        lab = lab_ref[pl.ds(i, 1), :]
        oh_t = (lax.broadcasted_iota(jnp.int32, (_C, _BA), 0)
                == lab).astype(jnp.float32)
        psum = lax.dot_general(oh_t, x, (((1,), (0,)), ((), ())),
                               preferred_element_type=jnp.float32)
        pcnt = jnp.sum(oh_t, axis=1, keepdims=True)

        cache_ref[pl.ds(i * _BA, _BA), :] = x.astype(jnp.bfloat16)

        @pl.when(i == 0)
        def _init():
            sums_ref[...] = psum
            cnt_ref[...] = pcnt

        @pl.when(i != 0)
        def _acc():
            sums_ref[...] += psum
            cnt_ref[...] += pcnt

    @pl.when(i >= _KA)
    def _phase_b():
        j = i - _KA

        @pl.when(i == _KA)
        def _finalize():
            cnt = cnt_ref[...]
            recip = 1.0 / jnp.maximum(cnt, 1.0)
            cent_ref[...] = (sums_ref[...] * recip).astype(jnp.bfloat16)
            bias_ref[...] = jnp.where(cnt > 0.0, 0.0, _NEG)

        cent = cent_ref[...]
        bias = bias_ref[...]
        lab = lab_ref[pl.ds(j, 1), :]
        oh_t = (lax.broadcasted_iota(jnp.int32, (_C, _BB), 0)
                == lab)

        def loss_of(x):
            lg = lax.dot_general(cent, x, (((1,), (1,)), ((), ())),
                                 preferred_element_type=jnp.float32)
            lg = lg * _INV_TEMP + bias
            m = jnp.max(lg, axis=0, keepdims=True)
            lse = jnp.log(jnp.sum(jnp.exp(lg - m), axis=0)) + m[0]
            picked = jnp.sum(jnp.where(oh_t, lg, 0.0), axis=0)
            return lse - picked

        x1 = cache_ref[pl.ds(j * _BB, _BB), :]
        x2 = s2_ref[...].astype(jnp.bfloat16)
        loss = loss_of(x1) + loss_of(x2)
        ones_row = jnp.full((1, 64), 1.0 / 64.0, dtype=jnp.float32)
        w = lax.dot_general(ones_row, sim_ref[0],
                            (((1,), (1,)), ((), ())),
                            preferred_element_type=jnp.float32)[0]
        part = jnp.sum(w * loss) * (0.25 / _N)

        @pl.when(i == _KA)
        def _out_init():
            out_ref[0, 0] = part

        @pl.when(i != _KA)
        def _out_acc():
            out_ref[0, 0] += part


def kernel(S1, S2, segmentation_map, similarity_matrix):
    out = pl.pallas_call(
        _fused,
        grid=(_KA + _KB,),
        in_specs=[
            pl.BlockSpec((_BA, _D), lambda i: (jnp.minimum(i, _KA - 1), 0)),
            pl.BlockSpec(memory_space=pltpu.VMEM),
            pl.BlockSpec((_BB, _D), lambda i: (jnp.maximum(i - _KA, 0), 0)),
            pl.BlockSpec((1, _BB, 64),
                         lambda i: (jnp.maximum(i - _KA, 0), 0, 0)),
        ],
        out_specs=pl.BlockSpec(memory_space=pltpu.SMEM),
        out_shape=jax.ShapeDtypeStruct((1, 1), jnp.float32),
        scratch_shapes=[
            pltpu.VMEM((_N, _D), jnp.bfloat16),
            pltpu.VMEM((_C, _D), jnp.float32),
            pltpu.VMEM((_C, 1), jnp.float32),
            pltpu.VMEM((_C, _D), jnp.bfloat16),
            pltpu.VMEM((_C, 1), jnp.float32),
        ],
        compiler_params=pltpu.CompilerParams(
            dimension_semantics=("arbitrary",)),
    )(S1, segmentation_map, S2, similarity_matrix)

    return out[0, 0]
